# Initial kernel scaffold; baseline (speedup 1.0000x reference)
import jax
import jax.numpy as jnp
from jax.experimental import pallas as pl


def kernel(x, W1, b1, W2, b2):
    raise NotImplementedError("write your pallas kernel here")


# speedup vs baseline: 127.5518x; 127.5518x over previous
import jax
import jax.numpy as jnp
from jax.experimental import pallas as pl
from jax.experimental.pallas import tpu as pltpu

_N = 10000
_NPAD = 10240
_DT = 512
_CH = 1024
_NT = _NPAD // _DT
_NCH = _NPAD // _CH
_THR = 0.3
_HID = 32


def _gnn_kernel(x_ref, w1_ref, b1_ref, w2_ref, b2_ref, out_ref,
                c_scr, ct_scr, ddr_scr, ddc_scr, v_scr, g_scr):
    phase = pl.program_id(0)
    t = pl.program_id(1)
    d0 = t * _DT

    @pl.when((phase == 0) & (t == 0))
    def _init():
        flat = x_ref[...].reshape(-1, _NPAD)
        ctr = flat.T
        mu = jnp.mean(ctr, axis=1, keepdims=True)
        ct = ctr - mu
        ct_scr[...] = ct
        c_scr[...] = ct.T
        v_scr[...] = jnp.mean(x_ref[:, x_ref.shape[1] - 1, :], axis=0,
                              keepdims=True)

        def dchunk(i, _):
            s0 = i * _CH
            blk = jax.lax.dot_general(
                ct_scr[pl.ds(s0, _CH), :],
                c_scr[:, pl.ds(s0, _CH)],
                (((1,), (0,)), ((), ())),
                preferred_element_type=jnp.float32)
            eye = (jax.lax.broadcasted_iota(jnp.int32, (_CH, _CH), 0) ==
                   jax.lax.broadcasted_iota(jnp.int32, (_CH, _CH), 1))
            dz = jnp.where(eye, blk, 0.0)
            fact = jnp.float32(flat.shape[0] - 1)
            ddr_scr[0:1, pl.ds(s0, _CH)] = jnp.sqrt(
                jnp.sum(dz, axis=0, keepdims=True) / fact)
            ddc_scr[pl.ds(s0, _CH), 0:1] = jnp.sqrt(
                jnp.sum(dz, axis=1, keepdims=True) / fact)
            return 0

        jax.lax.fori_loop(0, _NCH, dchunk, 0)

    cd = c_scr[:, pl.ds(d0, _DT)]
    ddd = ddr_scr[:, pl.ds(d0, _DT)]
    fact = jnp.float32(x_ref.shape[0] * x_ref.shape[1] - 1)

    def chunk(i, acc):
        s0 = i * _CH
        cts = ct_scr[pl.ds(s0, _CH), :]
        s = jax.lax.dot_general(cts, cd, (((1,), (0,)), ((), ())),
                                preferred_element_type=jnp.float32)
        dds = ddc_scr[pl.ds(s0, _CH), :]
        corr = ((s / fact) / dds) / ddd
        srow = jax.lax.broadcasted_iota(jnp.int32, (_CH, _DT), 0) + s0
        dcol = jax.lax.broadcasted_iota(jnp.int32, (_CH, _DT), 1) + d0
        m = (jnp.abs(corr) > _THR) & (srow != dcol)
        mf = m.astype(jnp.float32)
        vs = jnp.where(phase == 0,
                       v_scr[0:1, pl.ds(s0, _CH)],
                       g_scr[0:1, pl.ds(s0, _CH)])
        rows = jnp.concatenate([jnp.ones((1, _CH), jnp.float32), vs], axis=0)
        return acc + jax.lax.dot_general(rows, mf, (((1,), (0,)), ((), ())),
                                         precision=jax.lax.Precision.HIGHEST,
                                         preferred_element_type=jnp.float32)

    acc = jax.lax.fori_loop(0, _NCH, chunk,
                            jnp.zeros((2, _DT), jnp.float32))
    deg = acc[0:1, :] + 1.0
    num = acc[1:2, :]

    @pl.when(phase == 0)
    def _p0():
        vd = v_scr[0:1, pl.ds(d0, _DT)]
        a = (num + vd) / deg
        ab = jnp.broadcast_to(a, (_HID, _DT))
        h = jnp.maximum(ab * w1_ref[...] + b1_ref[...], 0.0)
        g = jnp.sum(h * w2_ref[...], axis=0, keepdims=True)
        g_scr[0:1, pl.ds(d0, _DT)] = g
        out_ref[...] = g

    @pl.when(phase == 1)
    def _p1():
        gd = g_scr[0:1, pl.ds(d0, _DT)]
        out_ref[...] = (num + gd) / deg + b2_ref[...]


@jax.jit
def _run(xpad, w1, b1, w2, b2):
    return pl.pallas_call(
        _gnn_kernel,
        grid=(2, _NT),
        in_specs=[
            pl.BlockSpec(xpad.shape, lambda p, t: (0, 0, 0)),
            pl.BlockSpec((_HID, 1), lambda p, t: (0, 0)),
            pl.BlockSpec((_HID, 1), lambda p, t: (0, 0)),
            pl.BlockSpec((_HID, 1), lambda p, t: (0, 0)),
            pl.BlockSpec((1, 1), lambda p, t: (0, 0)),
        ],
        out_specs=pl.BlockSpec((1, _DT), lambda p, t: (0, t)),
        out_shape=jax.ShapeDtypeStruct((1, _NPAD), jnp.float32),
        scratch_shapes=[
            pltpu.VMEM((64, _NPAD), jnp.float32),
            pltpu.VMEM((_NPAD, 64), jnp.float32),
            pltpu.VMEM((1, _NPAD), jnp.float32),
            pltpu.VMEM((_NPAD, 1), jnp.float32),
            pltpu.VMEM((1, _NPAD), jnp.float32),
            pltpu.VMEM((1, _NPAD), jnp.float32),
        ],
        compiler_params=pltpu.CompilerParams(
            dimension_semantics=("arbitrary", "arbitrary"),
        ),
    )(xpad, w1, b1, w2, b2)


def kernel(x, W1, b1, W2, b2):
    B, S, N = x.shape
    xpad = jnp.pad(x, ((0, 0), (0, 0), (0, _NPAD - N)))
    row = _run(xpad,
               W1.reshape(1, _HID).T,
               b1.reshape(_HID, 1),
               W2.reshape(_HID, 1),
               b2.reshape(1, 1))
    gnn = row[0, :N]
    return jnp.broadcast_to(gnn[None, None, :], (B, S, N))


# speedup vs baseline: 290.3783x; 2.2766x over previous
import jax
import jax.numpy as jnp
from jax.experimental import pallas as pl
from jax.experimental.pallas import tpu as pltpu

_N = 10000
_NPAD = 10240
_DT = 512
_CH = 1024
_NT = _NPAD // _DT
_NCH = _NPAD // _CH
_THR = 0.3
_HID = 32


def _gnn_kernel(x_ref, w1_ref, b1_ref, w2_ref, b2_ref, out_ref,
                c_scr, ct_scr, ddr_scr, ddc_scr, v_scr, g_scr):
    phase = pl.program_id(0)
    t = pl.program_id(1)
    d0 = t * _DT

    @pl.when((phase == 0) & (t == 0))
    def _init():
        flat = x_ref[...].reshape(-1, _NPAD)
        ctr = flat.T
        mu = jnp.mean(ctr, axis=1, keepdims=True)
        ct = ctr - mu
        ct_scr[...] = ct
        c_scr[...] = ct.T
        v_scr[...] = jnp.mean(x_ref[:, x_ref.shape[1] - 1, :], axis=0,
                              keepdims=True)

        def dchunk(i, _):
            s0 = i * _CH
            blk = jax.lax.dot_general(
                ct_scr[pl.ds(s0, _CH), :],
                c_scr[:, pl.ds(s0, _CH)],
                (((1,), (0,)), ((), ())),
                preferred_element_type=jnp.float32)
            eye = (jax.lax.broadcasted_iota(jnp.int32, (_CH, _CH), 0) ==
                   jax.lax.broadcasted_iota(jnp.int32, (_CH, _CH), 1))
            dz = jnp.where(eye, blk, 0.0)
            fact = jnp.float32(flat.shape[0] - 1)
            ddr_scr[0:1, pl.ds(s0, _CH)] = jnp.sqrt(
                jnp.sum(dz, axis=0, keepdims=True) / fact)
            ddc_scr[pl.ds(s0, _CH), 0:1] = jnp.sqrt(
                jnp.sum(dz, axis=1, keepdims=True) / fact)
            return 0

        jax.lax.fori_loop(0, _NCH, dchunk, 0)

    cd = c_scr[:, pl.ds(d0, _DT)]
    ddd = ddr_scr[:, pl.ds(d0, _DT)]
    fact = jnp.float32(x_ref.shape[0] * x_ref.shape[1] - 1)

    def chunk(i, acc):
        s0 = i * _CH
        cts = ct_scr[pl.ds(s0, _CH), :]
        s = jax.lax.dot_general(cts, cd, (((1,), (0,)), ((), ())),
                                preferred_element_type=jnp.float32)
        dds = ddc_scr[pl.ds(s0, _CH), :]
        corr = ((s / fact) / dds) / ddd
        mf = (jnp.abs(corr) > _THR).astype(jnp.float32)
        vs = jnp.where(phase == 0,
                       v_scr[0:1, pl.ds(s0, _CH)],
                       g_scr[0:1, pl.ds(s0, _CH)])
        rows = jnp.concatenate([jnp.ones((1, _CH), jnp.float32), vs], axis=0)
        return acc + jax.lax.dot_general(rows, mf, (((1,), (0,)), ((), ())),
                                         preferred_element_type=jnp.float32)

    acc = jax.lax.fori_loop(0, _NCH, chunk,
                            jnp.zeros((2, _DT), jnp.float32))
    deg0 = acc[0:1, :]
    isolated = deg0 == 0.0
    deg = jnp.where(isolated, 1.0, deg0)
    num = acc[1:2, :]

    @pl.when(phase == 0)
    def _p0():
        vd = v_scr[0:1, pl.ds(d0, _DT)]
        a = jnp.where(isolated, vd, num / deg)
        ab = jnp.broadcast_to(a, (_HID, _DT))
        h = jnp.maximum(ab * w1_ref[...] + b1_ref[...], 0.0)
        g = jnp.sum(h * w2_ref[...], axis=0, keepdims=True)
        g_scr[0:1, pl.ds(d0, _DT)] = g
        out_ref[...] = g

    @pl.when(phase == 1)
    def _p1():
        gd = g_scr[0:1, pl.ds(d0, _DT)]
        out_ref[...] = jnp.where(isolated, gd, num / deg) + b2_ref[...]


@jax.jit
def _run(xpad, w1, b1, w2, b2):
    return pl.pallas_call(
        _gnn_kernel,
        grid=(2, _NT),
        in_specs=[
            pl.BlockSpec(xpad.shape, lambda p, t: (0, 0, 0)),
            pl.BlockSpec((_HID, 1), lambda p, t: (0, 0)),
            pl.BlockSpec((_HID, 1), lambda p, t: (0, 0)),
            pl.BlockSpec((_HID, 1), lambda p, t: (0, 0)),
            pl.BlockSpec((1, 1), lambda p, t: (0, 0)),
        ],
        out_specs=pl.BlockSpec((1, _DT), lambda p, t: (0, t)),
        out_shape=jax.ShapeDtypeStruct((1, _NPAD), jnp.float32),
        scratch_shapes=[
            pltpu.VMEM((64, _NPAD), jnp.float32),
            pltpu.VMEM((_NPAD, 64), jnp.float32),
            pltpu.VMEM((1, _NPAD), jnp.float32),
            pltpu.VMEM((_NPAD, 1), jnp.float32),
            pltpu.VMEM((1, _NPAD), jnp.float32),
            pltpu.VMEM((1, _NPAD), jnp.float32),
        ],
        compiler_params=pltpu.CompilerParams(
            dimension_semantics=("arbitrary", "arbitrary"),
        ),
    )(xpad, w1, b1, w2, b2)


def kernel(x, W1, b1, W2, b2):
    B, S, N = x.shape
    xpad = jnp.pad(x, ((0, 0), (0, 0), (0, _NPAD - N)))
    row = _run(xpad,
               W1.reshape(1, _HID).T,
               b1.reshape(_HID, 1),
               W2.reshape(_HID, 1),
               b2.reshape(1, 1))
    gnn = row[0, :N]
    return jnp.broadcast_to(gnn[None, None, :], (B, S, N))


# speedup vs baseline: 340.8832x; 1.1739x over previous
import jax
import jax.numpy as jnp
from jax.experimental import pallas as pl
from jax.experimental.pallas import tpu as pltpu

_N = 10000
_NPAD = 10240
_DT = 512
_CH = 1024
_NT = _NPAD // _DT
_NCH = _NPAD // _CH
_THR = 0.3
_HID = 32


def _gnn_kernel(x_ref, w1_ref, b1_ref, w2_ref, b2_ref, out_ref,
                c_scr, ct_scr, ddr_scr, ddc_scr, v_scr, g_scr):
    phase = pl.program_id(0)
    t = pl.program_id(1)
    d0 = t * _DT

    @pl.when((phase == 0) & (t == 0))
    def _init():
        flat = x_ref[...].reshape(-1, _NPAD)
        ctr = flat.T
        mu = jnp.mean(ctr, axis=1, keepdims=True)
        ct = ctr - mu
        ct_scr[...] = ct
        c_scr[...] = ct.T
        v_scr[...] = jnp.mean(x_ref[:, x_ref.shape[1] - 1, :], axis=0,
                              keepdims=True)

        def dchunk(i, _):
            s0 = i * _CH
            blk = jax.lax.dot_general(
                ct_scr[pl.ds(s0, _CH), :],
                c_scr[:, pl.ds(s0, _CH)],
                (((1,), (0,)), ((), ())),
                preferred_element_type=jnp.float32)
            eye = (jax.lax.broadcasted_iota(jnp.int32, (_CH, _CH), 0) ==
                   jax.lax.broadcasted_iota(jnp.int32, (_CH, _CH), 1))
            dz = jnp.where(eye, blk, 0.0)
            fact = jnp.float32(flat.shape[0] - 1)
            scale = jnp.sqrt(jnp.float32(_THR) * fact)
            ddr_scr[0:1, pl.ds(s0, _CH)] = scale * jnp.sqrt(
                jnp.sum(dz, axis=0, keepdims=True) / fact)
            ddc_scr[pl.ds(s0, _CH), 0:1] = scale * jnp.sqrt(
                jnp.sum(dz, axis=1, keepdims=True) / fact)
            return 0

        jax.lax.fori_loop(0, _NCH, dchunk, 0)

    cd = c_scr[:, pl.ds(d0, _DT)]
    ddd = ddr_scr[:, pl.ds(d0, _DT)]
    fact = jnp.float32(x_ref.shape[0] * x_ref.shape[1] - 1)

    def chunk(i, acc):
        s0 = i * _CH
        cts = ct_scr[pl.ds(s0, _CH), :]
        s = jax.lax.dot_general(cts, cd, (((1,), (0,)), ((), ())),
                                preferred_element_type=jnp.float32)
        dds = ddc_scr[pl.ds(s0, _CH), :]
        mf = (jnp.abs(s) > dds * ddd).astype(jnp.float32)
        vs = jnp.where(phase == 0,
                       v_scr[0:1, pl.ds(s0, _CH)],
                       g_scr[0:1, pl.ds(s0, _CH)])
        rows = jnp.concatenate([jnp.ones((1, _CH), jnp.float32), vs], axis=0)
        return acc + jax.lax.dot_general(rows, mf, (((1,), (0,)), ((), ())),
                                         preferred_element_type=jnp.float32)

    acc = jax.lax.fori_loop(0, _NCH, chunk,
                            jnp.zeros((2, _DT), jnp.float32))
    deg0 = acc[0:1, :]
    isolated = deg0 == 0.0
    deg = jnp.where(isolated, 1.0, deg0)
    num = acc[1:2, :]

    @pl.when(phase == 0)
    def _p0():
        vd = v_scr[0:1, pl.ds(d0, _DT)]
        a = jnp.where(isolated, vd, num / deg)
        ab = jnp.broadcast_to(a, (_HID, _DT))
        h = jnp.maximum(ab * w1_ref[...] + b1_ref[...], 0.0)
        g = jnp.sum(h * w2_ref[...], axis=0, keepdims=True)
        g_scr[0:1, pl.ds(d0, _DT)] = g
        out_ref[...] = g

    @pl.when(phase == 1)
    def _p1():
        gd = g_scr[0:1, pl.ds(d0, _DT)]
        out_ref[...] = jnp.where(isolated, gd, num / deg) + b2_ref[...]


@jax.jit
def _run(xpad, w1, b1, w2, b2):
    return pl.pallas_call(
        _gnn_kernel,
        grid=(2, _NT),
        in_specs=[
            pl.BlockSpec(xpad.shape, lambda p, t: (0, 0, 0)),
            pl.BlockSpec((_HID, 1), lambda p, t: (0, 0)),
            pl.BlockSpec((_HID, 1), lambda p, t: (0, 0)),
            pl.BlockSpec((_HID, 1), lambda p, t: (0, 0)),
            pl.BlockSpec((1, 1), lambda p, t: (0, 0)),
        ],
        out_specs=pl.BlockSpec((1, _DT), lambda p, t: (0, t)),
        out_shape=jax.ShapeDtypeStruct((1, _NPAD), jnp.float32),
        scratch_shapes=[
            pltpu.VMEM((64, _NPAD), jnp.float32),
            pltpu.VMEM((_NPAD, 64), jnp.float32),
            pltpu.VMEM((1, _NPAD), jnp.float32),
            pltpu.VMEM((_NPAD, 1), jnp.float32),
            pltpu.VMEM((1, _NPAD), jnp.float32),
            pltpu.VMEM((1, _NPAD), jnp.float32),
        ],
        compiler_params=pltpu.CompilerParams(
            dimension_semantics=("arbitrary", "arbitrary"),
        ),
    )(xpad, w1, b1, w2, b2)


def kernel(x, W1, b1, W2, b2):
    B, S, N = x.shape
    xpad = jnp.pad(x, ((0, 0), (0, 0), (0, _NPAD - N)))
    row = _run(xpad,
               W1.reshape(1, _HID).T,
               b1.reshape(_HID, 1),
               W2.reshape(_HID, 1),
               b2.reshape(1, 1))
    gnn = row[0, :N]
    return jnp.broadcast_to(gnn[None, None, :], (B, S, N))


# speedup vs baseline: 374.3055x; 1.0980x over previous
import jax
import jax.numpy as jnp
from jax.experimental import pallas as pl
from jax.experimental.pallas import tpu as pltpu

_N = 10000
_NPAD = 10240
_DT = 1024
_CH = 1024
_NT = _NPAD // _DT
_NCH = _NPAD // _CH
_THR = 0.3
_HID = 32


def _gnn_kernel(x_ref, w1_ref, b1_ref, w2_ref, b2_ref, out_ref,
                c_scr, ct_scr, ddr_scr, ddc_scr, v_scr, g_scr):
    phase = pl.program_id(0)
    t = pl.program_id(1)
    d0 = t * _DT

    @pl.when((phase == 0) & (t == 0))
    def _init():
        flat = x_ref[...].reshape(-1, _NPAD)
        ctr = flat.T
        mu = jnp.mean(ctr, axis=1, keepdims=True)
        ct = ctr - mu
        ct_scr[...] = ct
        c_scr[...] = ct.T
        v_scr[...] = jnp.mean(x_ref[:, x_ref.shape[1] - 1, :], axis=0,
                              keepdims=True)

        def dchunk(i, _):
            s0 = i * _CH
            blk = jax.lax.dot_general(
                ct_scr[pl.ds(s0, _CH), :],
                c_scr[:, pl.ds(s0, _CH)],
                (((1,), (0,)), ((), ())),
                preferred_element_type=jnp.float32)
            eye = (jax.lax.broadcasted_iota(jnp.int32, (_CH, _CH), 0) ==
                   jax.lax.broadcasted_iota(jnp.int32, (_CH, _CH), 1))
            dz = jnp.where(eye, blk, 0.0)
            fact = jnp.float32(flat.shape[0] - 1)
            scale = jnp.sqrt(jnp.float32(_THR) * fact)
            ddr_scr[0:1, pl.ds(s0, _CH)] = scale * jnp.sqrt(
                jnp.sum(dz, axis=0, keepdims=True) / fact)
            ddc_scr[pl.ds(s0, _CH), 0:1] = scale * jnp.sqrt(
                jnp.sum(dz, axis=1, keepdims=True) / fact)
            return 0

        jax.lax.fori_loop(0, _NCH, dchunk, 0)

    cd = c_scr[:, pl.ds(d0, _DT)]
    ddd = ddr_scr[:, pl.ds(d0, _DT)]

    def chunk(i, acc):
        s0 = i * _CH
        cts = ct_scr[pl.ds(s0, _CH), :]
        s = jax.lax.dot_general(cts, cd, (((1,), (0,)), ((), ())),
                                preferred_element_type=jnp.float32)
        dds = ddc_scr[pl.ds(s0, _CH), :]
        mf = (jnp.abs(s) > dds * ddd).astype(jnp.float32)
        vs = jnp.where(phase == 0,
                       v_scr[0:1, pl.ds(s0, _CH)],
                       g_scr[0:1, pl.ds(s0, _CH)])
        rows = jnp.concatenate([jnp.ones((1, _CH), jnp.float32), vs], axis=0)
        return acc + jax.lax.dot_general(rows, mf, (((1,), (0,)), ((), ())),
                                         preferred_element_type=jnp.float32)

    acc = jax.lax.fori_loop(0, _NCH, chunk,
                            jnp.zeros((2, _DT), jnp.float32))
    deg0 = acc[0:1, :]
    isolated = deg0 == 0.0
    deg = jnp.where(isolated, 1.0, deg0)
    num = acc[1:2, :]

    @pl.when(phase == 0)
    def _p0():
        vd = v_scr[0:1, pl.ds(d0, _DT)]
        a = jnp.where(isolated, vd, num / deg)
        ab = jnp.broadcast_to(a, (_HID, _DT))
        h = jnp.maximum(ab * w1_ref[...] + b1_ref[...], 0.0)
        g = jnp.sum(h * w2_ref[...], axis=0, keepdims=True)
        g_scr[0:1, pl.ds(d0, _DT)] = g
        out_ref[...] = g

    @pl.when(phase == 1)
    def _p1():
        gd = g_scr[0:1, pl.ds(d0, _DT)]
        out_ref[...] = jnp.where(isolated, gd, num / deg) + b2_ref[...]


@jax.jit
def _run(xpad, w1, b1, w2, b2):
    return pl.pallas_call(
        _gnn_kernel,
        grid=(2, _NT),
        in_specs=[
            pl.BlockSpec(xpad.shape, lambda p, t: (0, 0, 0)),
            pl.BlockSpec((_HID, 1), lambda p, t: (0, 0)),
            pl.BlockSpec((_HID, 1), lambda p, t: (0, 0)),
            pl.BlockSpec((_HID, 1), lambda p, t: (0, 0)),
            pl.BlockSpec((1, 1), lambda p, t: (0, 0)),
        ],
        out_specs=pl.BlockSpec((1, _DT), lambda p, t: (0, t)),
        out_shape=jax.ShapeDtypeStruct((1, _NPAD), jnp.float32),
        scratch_shapes=[
            pltpu.VMEM((64, _NPAD), jnp.float32),
            pltpu.VMEM((_NPAD, 64), jnp.float32),
            pltpu.VMEM((1, _NPAD), jnp.float32),
            pltpu.VMEM((_NPAD, 1), jnp.float32),
            pltpu.VMEM((1, _NPAD), jnp.float32),
            pltpu.VMEM((1, _NPAD), jnp.float32),
        ],
        compiler_params=pltpu.CompilerParams(
            dimension_semantics=("arbitrary", "arbitrary"),
        ),
    )(xpad, w1, b1, w2, b2)


def kernel(x, W1, b1, W2, b2):
    B, S, N = x.shape
    xpad = jnp.pad(x, ((0, 0), (0, 0), (0, _NPAD - N)))
    row = _run(xpad,
               W1.reshape(1, _HID).T,
               b1.reshape(_HID, 1),
               W2.reshape(_HID, 1),
               b2.reshape(1, 1))
    gnn = row[0, :N]
    return jnp.broadcast_to(gnn[None, None, :], (B, S, N))


# speedup vs baseline: 382.8279x; 1.0228x over previous
import jax
import jax.numpy as jnp
from jax.experimental import pallas as pl
from jax.experimental.pallas import tpu as pltpu

_N = 10000
_NPAD = 10240
_DT = 1024
_CH = 1024
_NT = _NPAD // _DT
_NCH = _NPAD // _CH
_THR = 0.3
_HID = 32


def _gnn_kernel(x_ref, w1_ref, b1_ref, w2_ref, b2_ref, out_ref,
                c_scr, ct_scr, ddr_scr, ddc_scr, v_scr, g_scr,
                vc_scr, gc_scr, deg_scr):
    phase = pl.program_id(0)
    t = pl.program_id(1)
    d0 = t * _DT

    @pl.when((phase == 0) & (t == 0))
    def _init():
        flat = x_ref[...].reshape(-1, _NPAD)
        ctr = flat.T
        mu = jnp.mean(ctr, axis=1, keepdims=True)
        ct = ctr - mu
        ct_scr[...] = ct
        c_scr[...] = ct.T
        v = jnp.mean(x_ref[:, x_ref.shape[1] - 1, :], axis=0,
                     keepdims=True)
        v_scr[...] = v
        vc_scr[...] = jnp.broadcast_to(v, (8, _NPAD)).T[:, 0:1]

        def dchunk(i, _):
            s0 = i * _CH
            blk = jax.lax.dot_general(
                ct_scr[pl.ds(s0, _CH), :],
                c_scr[:, pl.ds(s0, _CH)],
                (((1,), (0,)), ((), ())),
                preferred_element_type=jnp.float32)
            eye = (jax.lax.broadcasted_iota(jnp.int32, (_CH, _CH), 0) ==
                   jax.lax.broadcasted_iota(jnp.int32, (_CH, _CH), 1))
            dz = jnp.where(eye, blk, 0.0)
            fact = jnp.float32(flat.shape[0] - 1)
            scale = jnp.sqrt(jnp.float32(_THR) * fact)
            ddr_scr[0:1, pl.ds(s0, _CH)] = scale * jnp.sqrt(
                jnp.sum(dz, axis=0, keepdims=True) / fact)
            ddc_scr[pl.ds(s0, _CH), 0:1] = scale * jnp.sqrt(
                jnp.sum(dz, axis=1, keepdims=True) / fact)
            return 0

        jax.lax.fori_loop(0, _NCH, dchunk, 0)

    cd = c_scr[:, pl.ds(d0, _DT)]
    ddd = ddr_scr[:, pl.ds(d0, _DT)]

    def block_mask(s0):
        cts = ct_scr[pl.ds(s0, _CH), :]
        s = jax.lax.dot_general(cts, cd, (((1,), (0,)), ((), ())),
                                preferred_element_type=jnp.float32)
        dds = ddc_scr[pl.ds(s0, _CH), :]
        return (jnp.abs(s) > dds * ddd).astype(jnp.float32)

    @pl.when(phase == 0)
    def _p0():
        def chunk(i, acc):
            s0 = i * _CH
            mf = block_mask(s0)
            vsc = vc_scr[pl.ds(s0, _CH), :]
            degp = jnp.sum(mf, axis=0, keepdims=True)
            nump = jnp.sum(mf * vsc, axis=0, keepdims=True)
            return acc + jnp.concatenate([degp, nump], axis=0)

        acc = jax.lax.fori_loop(0, _NCH, chunk,
                                jnp.zeros((2, _DT), jnp.float32))
        deg0 = acc[0:1, :]
        deg_scr[0:1, pl.ds(d0, _DT)] = deg0
        isolated = deg0 == 0.0
        vd = v_scr[0:1, pl.ds(d0, _DT)]
        a = jnp.where(isolated, vd, acc[1:2, :] / jnp.where(isolated, 1.0, deg0))
        ab = jnp.broadcast_to(a, (_HID, _DT))
        h = jnp.maximum(ab * w1_ref[...] + b1_ref[...], 0.0)
        g = jnp.sum(h * w2_ref[...], axis=0, keepdims=True)
        g_scr[0:1, pl.ds(d0, _DT)] = g
        gc_scr[pl.ds(d0, _DT), :] = jnp.broadcast_to(g, (8, _DT)).T[:, 0:1]
        out_ref[...] = g

    @pl.when(phase == 1)
    def _p1():
        def chunk(i, num):
            s0 = i * _CH
            mf = block_mask(s0)
            gsc = gc_scr[pl.ds(s0, _CH), :]
            return num + jnp.sum(mf * gsc, axis=0, keepdims=True)

        num = jax.lax.fori_loop(0, _NCH, chunk,
                                jnp.zeros((1, _DT), jnp.float32))
        deg0 = deg_scr[0:1, pl.ds(d0, _DT)]
        isolated = deg0 == 0.0
        gd = g_scr[0:1, pl.ds(d0, _DT)]
        out_ref[...] = jnp.where(
            isolated, gd, num / jnp.where(isolated, 1.0, deg0)) + b2_ref[...]


@jax.jit
def _run(xpad, w1, b1, w2, b2):
    return pl.pallas_call(
        _gnn_kernel,
        grid=(2, _NT),
        in_specs=[
            pl.BlockSpec(xpad.shape, lambda p, t: (0, 0, 0)),
            pl.BlockSpec((_HID, 1), lambda p, t: (0, 0)),
            pl.BlockSpec((_HID, 1), lambda p, t: (0, 0)),
            pl.BlockSpec((_HID, 1), lambda p, t: (0, 0)),
            pl.BlockSpec((1, 1), lambda p, t: (0, 0)),
        ],
        out_specs=pl.BlockSpec((1, _DT), lambda p, t: (0, t)),
        out_shape=jax.ShapeDtypeStruct((1, _NPAD), jnp.float32),
        scratch_shapes=[
            pltpu.VMEM((64, _NPAD), jnp.float32),
            pltpu.VMEM((_NPAD, 64), jnp.float32),
            pltpu.VMEM((1, _NPAD), jnp.float32),
            pltpu.VMEM((_NPAD, 1), jnp.float32),
            pltpu.VMEM((1, _NPAD), jnp.float32),
            pltpu.VMEM((1, _NPAD), jnp.float32),
            pltpu.VMEM((_NPAD, 1), jnp.float32),
            pltpu.VMEM((_NPAD, 1), jnp.float32),
            pltpu.VMEM((1, _NPAD), jnp.float32),
        ],
        compiler_params=pltpu.CompilerParams(
            dimension_semantics=("arbitrary", "arbitrary"),
        ),
    )(xpad, w1, b1, w2, b2)


def kernel(x, W1, b1, W2, b2):
    B, S, N = x.shape
    xpad = jnp.pad(x, ((0, 0), (0, 0), (0, _NPAD - N)))
    row = _run(xpad,
               W1.reshape(1, _HID).T,
               b1.reshape(_HID, 1),
               W2.reshape(_HID, 1),
               b2.reshape(1, 1))
    gnn = row[0, :N]
    return jnp.broadcast_to(gnn[None, None, :], (B, S, N))


# speedup vs baseline: 438.8837x; 1.1464x over previous
import jax
import jax.numpy as jnp
from jax.experimental import pallas as pl
from jax.experimental.pallas import tpu as pltpu

_N = 10000
_NPAD = 10240
_DT = 1024
_CH = 1024
_NT = _NPAD // _DT
_NCH = _NPAD // _CH
_THR = 0.3
_HID = 32


def _gnn_kernel(x_ref, w1_ref, b1_ref, w2_ref, b2_ref, out_ref,
                c_scr, ct_scr, ddr_scr, ddc_scr, v_scr, vc_scr,
                g_scr, gc_scr, deg_scr, pdeg_scr, pnum_scr,
                dcol_scr, ncol_scr):
    phase = pl.program_id(0)
    t = pl.program_id(1)
    d0 = t * _DT

    @pl.when((phase == 0) & (t == 0))
    def _init():
        flat = x_ref[...].reshape(-1, _NPAD)
        ctr = flat.T
        mu = jnp.mean(ctr, axis=1, keepdims=True)
        ct = ctr - mu
        ct_scr[...] = ct
        c_scr[...] = ct.T
        v = jnp.mean(x_ref[:, x_ref.shape[1] - 1, :], axis=0,
                     keepdims=True)
        v_scr[...] = v
        vc_scr[...] = jnp.broadcast_to(v, (8, _NPAD)).T[:, 0:1]

        def dchunk(i, _):
            s0 = i * _CH
            blk = jax.lax.dot_general(
                ct_scr[pl.ds(s0, _CH), :],
                c_scr[:, pl.ds(s0, _CH)],
                (((1,), (0,)), ((), ())),
                preferred_element_type=jnp.float32)
            eye = (jax.lax.broadcasted_iota(jnp.int32, (_CH, _CH), 0) ==
                   jax.lax.broadcasted_iota(jnp.int32, (_CH, _CH), 1))
            dz = jnp.where(eye, blk, 0.0)
            fact = jnp.float32(flat.shape[0] - 1)
            scale = jnp.sqrt(jnp.float32(_THR) * fact)
            ddr_scr[0:1, pl.ds(s0, _CH)] = scale * jnp.sqrt(
                jnp.sum(dz, axis=0, keepdims=True) / fact)
            ddc_scr[pl.ds(s0, _CH), 0:1] = scale * jnp.sqrt(
                jnp.sum(dz, axis=1, keepdims=True) / fact)
            return 0

        jax.lax.fori_loop(0, _NCH, dchunk, 0)

    @pl.when(t == 0)
    def _zero_cols():
        dcol_scr[...] = jnp.zeros((_NPAD, 1), jnp.float32)
        ncol_scr[...] = jnp.zeros((_NPAD, 1), jnp.float32)

    cd = c_scr[:, pl.ds(d0, _DT)]
    ddd = ddr_scr[:, pl.ds(d0, _DT)]
    valr = jnp.where(phase == 0,
                     v_scr[0:1, pl.ds(d0, _DT)],
                     g_scr[0:1, pl.ds(d0, _DT)])

    pdeg_scr[0:1, pl.ds(d0, _DT)] = jnp.zeros((1, _DT), jnp.float32)
    pnum_scr[0:1, pl.ds(d0, _DT)] = jnp.zeros((1, _DT), jnp.float32)

    def chunk(i, carry):
        @pl.when(i <= t)
        def _block():
            s0 = i * _CH
            cts = ct_scr[pl.ds(s0, _CH), :]
            s = jax.lax.dot_general(cts, cd, (((1,), (0,)), ((), ())),
                                    preferred_element_type=jnp.float32)
            dds = ddc_scr[pl.ds(s0, _CH), :]
            mf = (jnp.abs(s) > dds * ddd).astype(jnp.float32)
            vsc = jnp.where(phase == 0,
                            vc_scr[pl.ds(s0, _CH), :],
                            gc_scr[pl.ds(s0, _CH), :])
            pdeg_scr[0:1, pl.ds(d0, _DT)] += jnp.sum(
                mf, axis=0, keepdims=True)
            pnum_scr[0:1, pl.ds(d0, _DT)] += jnp.sum(
                mf * vsc, axis=0, keepdims=True)

            @pl.when(i < t)
            def _sym():
                dcol_scr[pl.ds(s0, _CH), 0:1] += jnp.sum(
                    mf, axis=1, keepdims=True)
                ncol_scr[pl.ds(s0, _CH), 0:1] += jnp.sum(
                    mf * valr, axis=1, keepdims=True)

        return carry

    jax.lax.fori_loop(0, _NCH, chunk, 0)

    @pl.when((phase == 0) & (t == _NT - 1))
    def _epilogue0():
        dcol_row = jnp.broadcast_to(dcol_scr[...], (_NPAD, 8)).T[0:1, :]
        ncol_row = jnp.broadcast_to(ncol_scr[...], (_NPAD, 8)).T[0:1, :]
        deg0 = pdeg_scr[...] + dcol_row
        num = pnum_scr[...] + ncol_row
        deg_scr[...] = deg0
        isolated = deg0 == 0.0
        a = jnp.where(isolated, v_scr[...],
                      num / jnp.where(isolated, 1.0, deg0))
        ab = jnp.broadcast_to(a, (_HID, _NPAD))
        h = jnp.maximum(ab * w1_ref[...] + b1_ref[...], 0.0)
        g = jnp.sum(h * w2_ref[...], axis=0, keepdims=True)
        g_scr[...] = g
        gc_scr[...] = jnp.broadcast_to(g, (8, _NPAD)).T[:, 0:1]

    @pl.when((phase == 1) & (t == _NT - 1))
    def _epilogue1():
        ncol_row = jnp.broadcast_to(ncol_scr[...], (_NPAD, 8)).T[0:1, :]
        num = pnum_scr[...] + ncol_row
        deg0 = deg_scr[...]
        isolated = deg0 == 0.0
        out_ref[...] = jnp.where(
            isolated, g_scr[...],
            num / jnp.where(isolated, 1.0, deg0)) + b2_ref[...]


@jax.jit
def _run(xpad, w1, b1, w2, b2):
    return pl.pallas_call(
        _gnn_kernel,
        grid=(2, _NT),
        in_specs=[
            pl.BlockSpec(xpad.shape, lambda p, t: (0, 0, 0)),
            pl.BlockSpec((_HID, 1), lambda p, t: (0, 0)),
            pl.BlockSpec((_HID, 1), lambda p, t: (0, 0)),
            pl.BlockSpec((_HID, 1), lambda p, t: (0, 0)),
            pl.BlockSpec((1, 1), lambda p, t: (0, 0)),
        ],
        out_specs=pl.BlockSpec((1, _NPAD), lambda p, t: (0, 0)),
        out_shape=jax.ShapeDtypeStruct((1, _NPAD), jnp.float32),
        scratch_shapes=[
            pltpu.VMEM((64, _NPAD), jnp.float32),
            pltpu.VMEM((_NPAD, 64), jnp.float32),
            pltpu.VMEM((1, _NPAD), jnp.float32),
            pltpu.VMEM((_NPAD, 1), jnp.float32),
            pltpu.VMEM((1, _NPAD), jnp.float32),
            pltpu.VMEM((_NPAD, 1), jnp.float32),
            pltpu.VMEM((1, _NPAD), jnp.float32),
            pltpu.VMEM((_NPAD, 1), jnp.float32),
            pltpu.VMEM((1, _NPAD), jnp.float32),
            pltpu.VMEM((1, _NPAD), jnp.float32),
            pltpu.VMEM((1, _NPAD), jnp.float32),
            pltpu.VMEM((_NPAD, 1), jnp.float32),
            pltpu.VMEM((_NPAD, 1), jnp.float32),
        ],
        compiler_params=pltpu.CompilerParams(
            dimension_semantics=("arbitrary", "arbitrary"),
        ),
    )(xpad, w1, b1, w2, b2)


def kernel(x, W1, b1, W2, b2):
    B, S, N = x.shape
    xpad = jnp.pad(x, ((0, 0), (0, 0), (0, _NPAD - N)))
    row = _run(xpad,
               W1.reshape(1, _HID).T,
               b1.reshape(_HID, 1),
               W2.reshape(_HID, 1),
               b2.reshape(1, 1))
    gnn = row[0, :N]
    return jnp.broadcast_to(gnn[None, None, :], (B, S, N))
